# baseline (speedup 1.0000x reference)
import jax
import jax.numpy as jnp
from jax import lax
from jax.experimental import pallas as pl
from jax.experimental.pallas import tpu as pltpu

_F32 = jnp.float32
_BF16 = jnp.bfloat16
_HI = lax.Precision.HIGHEST


def _dot(a, b):
    return jnp.dot(a, b, precision=_HI, preferred_element_type=_F32)


def _sigmoid(x):
    return 0.5 + 0.5 * jnp.tanh(0.5 * x)


_OFF = dict(wih=0, whh=192, bih=256, bhh=257, w2l=258, w2r=274, b2=290,
            hx=291, cx=292, w1l=293, b1=294, w1r=295, w0=296, b0=680,
            wa=681, ba=713, wb=714, bb=730, wc=731, bc=739, x=740)
_ROWS = 804


def _net_body(rti_ref, edge_ref, e2ni_ref, e2n_ref, pack_ref,
              out_ref, gi_s, gf_s, gg_s, go_s, wi_s, wf_s, wg_s, wo_s):
    o_ = _OFF

    niota = lax.broadcasted_iota(jnp.int32, (64, 2048), 0)
    src_oh_t = (edge_ref[0:1, :] == niota).astype(_BF16)
    dst_oh_t = (edge_ref[1:2, :] == niota).astype(_BF16)
    a_cnt = lax.dot_general(dst_oh_t, src_oh_t,
                            (((1,), (1,)), ((), ())),
                            preferred_element_type=_F32)
    inv_cnt = 1.0 / jnp.maximum(jnp.sum(a_cnt, axis=1, keepdims=True), 1.0)

    x = pack_ref[o_["x"]:o_["x"] + 64, 0:1]
    agg1 = _dot(a_cnt, x) * inv_cnt
    h1 = jnp.maximum(agg1 * pack_ref[o_["w1l"]:o_["w1l"] + 1, 0:16]
                     + x * pack_ref[o_["w1r"]:o_["w1r"] + 1, 0:16]
                     + pack_ref[o_["b1"]:o_["b1"] + 1, 0:16], 0.0)

    agg2 = _dot(a_cnt, h1) * inv_cnt
    h2 = jnp.maximum(_dot(agg2, pack_ref[o_["w2l"]:o_["w2l"] + 16, 0:64])
                     + _dot(h1, pack_ref[o_["w2r"]:o_["w2r"] + 16, 0:64])
                     + pack_ref[o_["b2"]:o_["b2"] + 1, 0:64], 0.0)
    x_g = jnp.sum(h2, axis=0, keepdims=True) * (1.0 / 64.0)

    oh_idx = (e2ni_ref[...].reshape(64, 1)
              == lax.broadcasted_iota(jnp.int32, (64, 128), 1)
              ).astype(_BF16)
    v_iota = lax.broadcasted_iota(jnp.int32, (128, 64), 1)
    e2n = e2n_ref[...]
    e2ns_oh = (e2n[:, 0:1] == v_iota).astype(_BF16)
    e2nt_oh = (e2n[:, 1:2] == v_iota).astype(_BF16)
    p0 = jnp.dot(oh_idx, e2ns_oh, preferred_element_type=_F32)
    p1 = jnp.dot(oh_idx, e2nt_oh, preferred_element_type=_F32)

    seq = jnp.concatenate([h2, p0, p1], axis=1)
    wih = pack_ref[o_["wih"]:o_["wih"] + 192, :]
    bihh = (pack_ref[o_["bih"]:o_["bih"] + 1, :]
            + pack_ref[o_["bhh"]:o_["bhh"] + 1, :])
    gi_s[...] = _dot(seq, wih[:, 0:64]) + bihh[:, 0:64]
    gf_s[...] = _dot(seq, wih[:, 64:128]) + bihh[:, 64:128]
    gg_s[...] = _dot(seq, wih[:, 128:192]) + bihh[:, 128:192]
    go_s[...] = _dot(seq, wih[:, 192:256]) + bihh[:, 192:256]

    whh = pack_ref[o_["whh"]:o_["whh"] + 64, :]
    wi_s[...] = whh[:, 0:64]
    wf_s[...] = whh[:, 64:128]
    wg_s[...] = whh[:, 128:192]
    wo_s[...] = whh[:, 192:256]
    whh_i = wi_s[...]
    whh_f = wf_s[...]
    whh_g = wg_s[...]
    whh_o = wo_s[...]
    hh = pack_ref[o_["hx"]:o_["hx"] + 1, 0:64]
    cc = pack_ref[o_["cx"]:o_["cx"] + 1, 0:64]
    for t in range(64):
        hh_c = hh.reshape(64, 1)
        ri = jnp.sum(hh_c * whh_i, axis=0, keepdims=True)
        rf = jnp.sum(hh_c * whh_f, axis=0, keepdims=True)
        rg = jnp.sum(hh_c * whh_g, axis=0, keepdims=True)
        ro = jnp.sum(hh_c * whh_o, axis=0, keepdims=True)
        i_t = _sigmoid(gi_s[t:t + 1, :] + ri)
        f_t = _sigmoid(gf_s[t:t + 1, :] + rf)
        o_t = _sigmoid(go_s[t:t + 1, :] + ro)
        g_t = jnp.tanh(gg_s[t:t + 1, :] + rg)
        cc = f_t * cc + i_t * g_t
        hh = o_t * jnp.tanh(cc)

    lane = lax.broadcasted_iota(jnp.int32, (1, 64), 1)
    s_oh = (lane == rti_ref[0]).astype(_F32)
    p_oh = (lane == rti_ref[1]).astype(_F32)
    d_oh = (lane == rti_ref[2]).astype(_F32)
    feat = jnp.concatenate([cc, hh, x_g, s_oh, p_oh, d_oh], axis=1)

    o = jnp.maximum(_dot(feat, pack_ref[o_["w0"]:o_["w0"] + 384, 0:32])
                    + pack_ref[o_["b0"]:o_["b0"] + 1, 0:32], 0.0)
    o = jnp.maximum(_dot(o, pack_ref[o_["wa"]:o_["wa"] + 32, 0:16])
                    + pack_ref[o_["ba"]:o_["ba"] + 1, 0:16], 0.0)
    o = jnp.maximum(_dot(o, pack_ref[o_["wb"]:o_["wb"] + 16, 0:8])
                    + pack_ref[o_["bb"]:o_["bb"] + 1, 0:8], 0.0)
    o = jnp.maximum(_dot(o, pack_ref[o_["wc"]:o_["wc"] + 8, 0:1])
                    + pack_ref[o_["bc"]:o_["bc"] + 1, 0:1], 0.0)
    out_ref[...] = o


def _padw(a):
    return jnp.pad(a, ((0, 0), (0, 256 - a.shape[1])))


def kernel(x, edge_index, edge_to_node_index, edge_to_node, routing_table_item,
           hx, cx, W1l, b1, W1r, W2l, b2, W2r, Wih, bih, Whh, bhh,
           W0, b0, Wa, ba, Wb, bb, Wc, bc):
    pack = jnp.concatenate([
        Wih, Whh, bih.reshape(1, 256), bhh.reshape(1, 256),
        _padw(W2l), _padw(W2r), _padw(b2.reshape(1, 64)),
        _padw(hx.reshape(1, 64)), _padw(cx.reshape(1, 64)),
        _padw(W1l), _padw(b1.reshape(1, 16)), _padw(W1r),
        _padw(W0), _padw(b0.reshape(1, 32)),
        _padw(Wa), _padw(ba.reshape(1, 16)),
        _padw(Wb), _padw(bb.reshape(1, 8)),
        _padw(Wc), _padw(bc.reshape(1, 1)),
        _padw(x),
    ], axis=0)
    args = (
        routing_table_item,
        edge_index,
        edge_to_node_index.reshape(1, 64),
        edge_to_node,
        pack,
    )
    in_specs = ([pl.BlockSpec(memory_space=pltpu.SMEM)]
                + [pl.BlockSpec(memory_space=pltpu.VMEM)] * 4)
    out = pl.pallas_call(
        _net_body,
        out_shape=jax.ShapeDtypeStruct((1, 1), jnp.float32),
        in_specs=in_specs,
        out_specs=pl.BlockSpec(memory_space=pltpu.VMEM),
        scratch_shapes=[pltpu.VMEM((64, 64), jnp.float32)] * 8,
    )(*args)
    return out.reshape(1)


# speedup vs baseline: 2.4527x; 2.4527x over previous
import jax
import jax.numpy as jnp
from jax import lax
from jax.experimental import pallas as pl
from jax.experimental.pallas import tpu as pltpu

_F32 = jnp.float32
_BF16 = jnp.bfloat16
_HI = lax.Precision.HIGHEST


def _dot(a, b):
    return jnp.dot(a, b, precision=_HI, preferred_element_type=_F32)


def _sigmoid(x):
    return 0.5 + 0.5 * jnp.tanh(0.5 * x)


def _net_body(rti_ref, edge_ref, e2ni_ref, e2n_ref,
              x_ref, hx_ref, cx_ref,
              w1l_ref, b1_ref, w1r_ref, w2l_ref, b2_ref, w2r_ref,
              wih_ref, bih_ref, whh_ref, bhh_ref,
              w0_ref, b0_ref, wa_ref, ba_ref, wb_ref, bb_ref, wc_ref, bc_ref,
              out_ref, gi_s, gf_s, gg_s, go_s, wi_s, wf_s, wg_s, wo_s):
    niota = lax.broadcasted_iota(jnp.int32, (64, 2048), 0)
    src_oh_t = (edge_ref[0:1, :] == niota).astype(_BF16)
    dst_oh_t = (edge_ref[1:2, :] == niota).astype(_BF16)
    a_cnt = lax.dot_general(dst_oh_t, src_oh_t,
                            (((1,), (1,)), ((), ())),
                            preferred_element_type=_F32)
    inv_cnt = 1.0 / jnp.maximum(jnp.sum(a_cnt, axis=1, keepdims=True), 1.0)

    x = x_ref[...]
    agg1 = _dot(a_cnt, x) * inv_cnt
    h1 = jnp.maximum(agg1 * w1l_ref[...] + x * w1r_ref[...] + b1_ref[...], 0.0)

    agg2 = _dot(a_cnt, h1) * inv_cnt
    h2 = jnp.maximum(_dot(agg2, w2l_ref[...]) + _dot(h1, w2r_ref[...])
                     + b2_ref[...], 0.0)
    x_g = jnp.sum(h2, axis=0, keepdims=True) * (1.0 / 64.0)

    oh_idx = (e2ni_ref[...].reshape(64, 1)
              == lax.broadcasted_iota(jnp.int32, (64, 128), 1)
              ).astype(_BF16)
    v_iota = lax.broadcasted_iota(jnp.int32, (128, 64), 1)
    e2n = e2n_ref[...]
    e2ns_oh = (e2n[:, 0:1] == v_iota).astype(_BF16)
    e2nt_oh = (e2n[:, 1:2] == v_iota).astype(_BF16)
    p0 = jnp.dot(oh_idx, e2ns_oh, preferred_element_type=_F32)
    p1 = jnp.dot(oh_idx, e2nt_oh, preferred_element_type=_F32)

    seq = jnp.concatenate([h2, p0, p1], axis=1)
    wih = wih_ref[...]
    bihh = bih_ref[...] + bhh_ref[...]
    gi_s[...] = _dot(seq, wih[:, 0:64]) + bihh[:, 0:64]
    gf_s[...] = _dot(seq, wih[:, 64:128]) + bihh[:, 64:128]
    gg_s[...] = _dot(seq, wih[:, 128:192]) + bihh[:, 128:192]
    go_s[...] = _dot(seq, wih[:, 192:256]) + bihh[:, 192:256]

    whh = whh_ref[...]
    wi_s[...] = whh[:, 0:64]
    wf_s[...] = whh[:, 64:128]
    wg_s[...] = whh[:, 128:192]
    wo_s[...] = whh[:, 192:256]
    whh_i = wi_s[...]
    whh_f = wf_s[...]
    whh_g = wg_s[...]
    whh_o = wo_s[...]
    hh = hx_ref[...]
    cc = cx_ref[...]
    for t in range(64):
        hh_c = hh.reshape(64, 1)
        ri = jnp.sum(hh_c * whh_i, axis=0, keepdims=True)
        rf = jnp.sum(hh_c * whh_f, axis=0, keepdims=True)
        rg = jnp.sum(hh_c * whh_g, axis=0, keepdims=True)
        ro = jnp.sum(hh_c * whh_o, axis=0, keepdims=True)
        i_t = _sigmoid(gi_s[t:t + 1, :] + ri)
        f_t = _sigmoid(gf_s[t:t + 1, :] + rf)
        o_t = _sigmoid(go_s[t:t + 1, :] + ro)
        g_t = jnp.tanh(gg_s[t:t + 1, :] + rg)
        cc = f_t * cc + i_t * g_t
        hh = o_t * jnp.tanh(cc)

    lane = lax.broadcasted_iota(jnp.int32, (1, 64), 1)
    s_oh = (lane == rti_ref[0]).astype(_F32)
    p_oh = (lane == rti_ref[1]).astype(_F32)
    d_oh = (lane == rti_ref[2]).astype(_F32)
    feat = jnp.concatenate([cc, hh, x_g, s_oh, p_oh, d_oh], axis=1)

    o = jnp.maximum(_dot(feat, w0_ref[...]) + b0_ref[...], 0.0)
    o = jnp.maximum(_dot(o, wa_ref[...]) + ba_ref[...], 0.0)
    o = jnp.maximum(_dot(o, wb_ref[...]) + bb_ref[...], 0.0)
    o = jnp.maximum(_dot(o, wc_ref[...]) + bc_ref[...], 0.0)
    out_ref[...] = o


def kernel(x, edge_index, edge_to_node_index, edge_to_node, routing_table_item,
           hx, cx, W1l, b1, W1r, W2l, b2, W2r, Wih, bih, Whh, bhh,
           W0, b0, Wa, ba, Wb, bb, Wc, bc):
    args = (
        routing_table_item,
        edge_index,
        edge_to_node_index.reshape(1, 64),
        edge_to_node,
        x,
        hx.reshape(1, 64),
        cx.reshape(1, 64),
        W1l, b1.reshape(1, 16), W1r,
        W2l, b2.reshape(1, 64), W2r,
        Wih, bih.reshape(1, 256), Whh, bhh.reshape(1, 256),
        W0, b0.reshape(1, 32), Wa, ba.reshape(1, 16),
        Wb, bb.reshape(1, 8), Wc, bc.reshape(1, 1),
    )
    in_specs = ([pl.BlockSpec(memory_space=pltpu.SMEM)]
                + [pl.BlockSpec(memory_space=pltpu.VMEM)] * (len(args) - 1))
    out = pl.pallas_call(
        _net_body,
        out_shape=jax.ShapeDtypeStruct((1, 1), jnp.float32),
        in_specs=in_specs,
        out_specs=pl.BlockSpec(memory_space=pltpu.VMEM),
        scratch_shapes=[pltpu.VMEM((64, 64), jnp.float32)] * 8,
    )(*args)
    return out.reshape(1)


# speedup vs baseline: 2.5059x; 1.0217x over previous
import jax
import jax.numpy as jnp
from jax import lax
from jax.experimental import pallas as pl
from jax.experimental.pallas import tpu as pltpu

_F32 = jnp.float32
_BF16 = jnp.bfloat16
_HI = lax.Precision.HIGHEST


def _dot(a, b):
    return jnp.dot(a, b, precision=_HI, preferred_element_type=_F32)


def _dotd(a, b):
    return jnp.dot(a.astype(_BF16), b.astype(_BF16),
                   preferred_element_type=_F32)


def _rnd(a):
    return a.astype(_BF16).astype(_F32)


def _sigmoid(x):
    return 0.5 + 0.5 * jnp.tanh(0.5 * x)


def _net_body(rti_ref, edge_ref, e2ni_ref, e2n_ref,
              x_ref, hx_ref, cx_ref,
              w1l_ref, b1_ref, w1r_ref, w2l_ref, b2_ref, w2r_ref,
              wih_ref, bih_ref, whh_ref, bhh_ref,
              w0_ref, b0_ref, wa_ref, ba_ref, wb_ref, bb_ref, wc_ref, bc_ref,
              out_ref, gi_s, gf_s, gg_s, go_s, wi_s, wf_s, wg_s, wo_s):
    niota = lax.broadcasted_iota(jnp.int32, (64, 2048), 0)
    src_oh_t = (edge_ref[0:1, :] == niota).astype(_BF16)
    dst_oh_t = (edge_ref[1:2, :] == niota).astype(_BF16)
    a_cnt = lax.dot_general(dst_oh_t, src_oh_t,
                            (((1,), (1,)), ((), ())),
                            preferred_element_type=_F32)
    inv_cnt = 1.0 / jnp.maximum(jnp.sum(a_cnt, axis=1, keepdims=True), 1.0)

    x = x_ref[...]
    agg1 = _dot(a_cnt, x) * inv_cnt
    h1 = jnp.maximum(_rnd(agg1) * _rnd(w1l_ref[...])
                     + _rnd(x) * _rnd(w1r_ref[...]) + b1_ref[...], 0.0)

    agg2 = _dot(a_cnt, h1) * inv_cnt
    h2 = jnp.maximum(_dotd(agg2, w2l_ref[...]) + _dotd(h1, w2r_ref[...])
                     + b2_ref[...], 0.0)
    x_g = jnp.sum(h2, axis=0, keepdims=True) * (1.0 / 64.0)

    oh_idx = (e2ni_ref[...].reshape(64, 1)
              == lax.broadcasted_iota(jnp.int32, (64, 128), 1)
              ).astype(_BF16)
    v_iota = lax.broadcasted_iota(jnp.int32, (128, 64), 1)
    e2n = e2n_ref[...]
    e2ns_oh = (e2n[:, 0:1] == v_iota).astype(_BF16)
    e2nt_oh = (e2n[:, 1:2] == v_iota).astype(_BF16)
    p0 = jnp.dot(oh_idx, e2ns_oh, preferred_element_type=_F32)
    p1 = jnp.dot(oh_idx, e2nt_oh, preferred_element_type=_F32)

    seq = jnp.concatenate([h2, p0, p1], axis=1)
    wih = wih_ref[...]
    bihh = bih_ref[...] + bhh_ref[...]
    gi_s[...] = _dotd(seq, wih[:, 0:64]) + bihh[:, 0:64]
    gf_s[...] = _dotd(seq, wih[:, 64:128]) + bihh[:, 64:128]
    gg_s[...] = _dotd(seq, wih[:, 128:192]) + bihh[:, 128:192]
    go_s[...] = _dotd(seq, wih[:, 192:256]) + bihh[:, 192:256]

    whh = whh_ref[...]
    whh = _rnd(whh)
    wi_s[...] = whh[:, 0:64]
    wf_s[...] = whh[:, 64:128]
    wg_s[...] = whh[:, 128:192]
    wo_s[...] = whh[:, 192:256]
    whh_i = wi_s[...]
    whh_f = wf_s[...]
    whh_g = wg_s[...]
    whh_o = wo_s[...]
    hh = hx_ref[...]
    cc = cx_ref[...]
    for t in range(64):
        hh_c = _rnd(hh).reshape(64, 1)
        ri = jnp.sum(hh_c * whh_i, axis=0, keepdims=True)
        rf = jnp.sum(hh_c * whh_f, axis=0, keepdims=True)
        rg = jnp.sum(hh_c * whh_g, axis=0, keepdims=True)
        ro = jnp.sum(hh_c * whh_o, axis=0, keepdims=True)
        i_t = _sigmoid(gi_s[t:t + 1, :] + ri)
        f_t = _sigmoid(gf_s[t:t + 1, :] + rf)
        o_t = _sigmoid(go_s[t:t + 1, :] + ro)
        g_t = jnp.tanh(gg_s[t:t + 1, :] + rg)
        cc = f_t * cc + i_t * g_t
        hh = o_t * jnp.tanh(cc)

    lane = lax.broadcasted_iota(jnp.int32, (1, 64), 1)
    s_oh = (lane == rti_ref[0]).astype(_F32)
    p_oh = (lane == rti_ref[1]).astype(_F32)
    d_oh = (lane == rti_ref[2]).astype(_F32)
    feat = jnp.concatenate([cc, hh, x_g, s_oh, p_oh, d_oh], axis=1)

    o = jnp.maximum(_dotd(feat, w0_ref[...]) + b0_ref[...], 0.0)
    o = jnp.maximum(_dotd(o, wa_ref[...]) + ba_ref[...], 0.0)
    o = jnp.maximum(_dotd(o, wb_ref[...]) + bb_ref[...], 0.0)
    o = jnp.maximum(_dotd(o, wc_ref[...]) + bc_ref[...], 0.0)
    out_ref[...] = o


def kernel(x, edge_index, edge_to_node_index, edge_to_node, routing_table_item,
           hx, cx, W1l, b1, W1r, W2l, b2, W2r, Wih, bih, Whh, bhh,
           W0, b0, Wa, ba, Wb, bb, Wc, bc):
    args = (
        routing_table_item,
        edge_index,
        edge_to_node_index.reshape(1, 64),
        edge_to_node,
        x,
        hx.reshape(1, 64),
        cx.reshape(1, 64),
        W1l, b1.reshape(1, 16), W1r,
        W2l, b2.reshape(1, 64), W2r,
        Wih, bih.reshape(1, 256), Whh, bhh.reshape(1, 256),
        W0, b0.reshape(1, 32), Wa, ba.reshape(1, 16),
        Wb, bb.reshape(1, 8), Wc, bc.reshape(1, 1),
    )
    in_specs = ([pl.BlockSpec(memory_space=pltpu.SMEM)]
                + [pl.BlockSpec(memory_space=pltpu.VMEM)] * (len(args) - 1))
    out = pl.pallas_call(
        _net_body,
        out_shape=jax.ShapeDtypeStruct((1, 1), jnp.float32),
        in_specs=in_specs,
        out_specs=pl.BlockSpec(memory_space=pltpu.VMEM),
        scratch_shapes=[pltpu.VMEM((64, 64), jnp.float32)] * 8,
    )(*args)
    return out.reshape(1)
